# baseline (speedup 1.0000x reference)
import functools

import jax
import jax.numpy as jnp
from jax import lax
from jax.experimental import pallas as pl
from jax.experimental.pallas import tpu as pltpu
from jax.experimental.pallas import tpu_sc as plsc

N = 10000
NP = 10240
E = 320000
EP = 327680
D_IN = 128
NHID = 256
NCLASS = 40
NCP = 64

NC = 2
NS = 16
NHBM = 6
EB = 128
ROWS_PER_TILE = NP // NS

f32 = jnp.float32
i32 = jnp.int32


def _sc_mesh():
    return plsc.VectorSubcoreMesh(
        core_axis_name="c", subcore_axis_name="s", num_cores=NC, num_subcores=NS
    )


def _make_deg_kernel():
    ept = EP // (NC * NS)
    nb = ept // EB

    @functools.partial(
        pl.kernel,
        out_type=[
            jax.ShapeDtypeStruct((NP, 16), f32),
            jax.ShapeDtypeStruct((NP, 16), f32),
        ],
        mesh=_sc_mesh(),
        scratch_types=[
            pltpu.VMEM((EP // (NC * NS * EB), EB), i32),
            pltpu.VMEM((EB, 16), f32),
            pltpu.VMEM_SHARED((NP, 16), f32),
        ],
        compiler_params=pltpu.CompilerParams(use_tc_tiling_on_sc=False),
    )
    def deg_kernel(dst2d_hbm, ones_hbm, deg0_hbm, deg1_hbm, dst_v, ones_v, acc_sh):
        c = lax.axis_index("c")
        s = lax.axis_index("s")
        r0 = s * ROWS_PER_TILE
        pltpu.sync_copy(ones_hbm, acc_sh.at[pl.ds(r0, ROWS_PER_TILE)])
        pltpu.sync_copy(ones_hbm.at[pl.ds(0, EB)], ones_v)
        row0 = (c * NS + s) * nb
        pltpu.sync_copy(dst2d_hbm.at[pl.ds(row0, nb)], dst_v)
        plsc.subcore_barrier()

        def body(i):
            pltpu.sync_copy(ones_v, acc_sh.at[dst_v.at[i]], add=True)

        pl.loop(0, nb)(body)
        plsc.subcore_barrier()

        @pl.when(c == 0)
        def _():
            pltpu.sync_copy(
                acc_sh.at[pl.ds(r0, ROWS_PER_TILE)],
                deg0_hbm.at[pl.ds(r0, ROWS_PER_TILE)],
            )

        @pl.when(c == 1)
        def _():
            pltpu.sync_copy(
                acc_sh.at[pl.ds(r0, ROWS_PER_TILE)],
                deg1_hbm.at[pl.ds(r0, ROWS_PER_TILE)],
            )

    return deg_kernel


def _make_agg_kernel(dh, narr):
    ept = EP // NS
    nb = ept // EB
    ch = 32
    nch = nb // ch
    ngrp = ch // 4

    @functools.partial(
        pl.kernel,
        out_type=[jax.ShapeDtypeStruct((NP, 2 * dh), f32)] * narr,
        mesh=_sc_mesh(),
        scratch_types=[
            pltpu.VMEM((ch, EB), i32),
            pltpu.VMEM((ch, EB), i32),
            [pltpu.VMEM((EB, dh), f32)] * 4,
            pltpu.VMEM_SHARED((NP, dh), f32),
            pltpu.VMEM_SHARED((NP, dh), f32),
            [pltpu.SemaphoreType.DMA] * 4,
            [pltpu.SemaphoreType.DMA] * 4,
        ],
        compiler_params=pltpu.CompilerParams(use_tc_tiling_on_sc=False),
    )
    def agg_kernel(*refs):
        hps = refs[:narr]
        hp2s = refs[narr : 2 * narr]
        src2d, srca, srcb, dst2d = refs[2 * narr : 2 * narr + 4]
        outs = refs[2 * narr + 4 : 3 * narr + 4]
        src_v, dst_v, bufs, acc_sh, tbl_sh, gsems, ssems = refs[3 * narr + 4 :]
        c = lax.axis_index("c")
        s = lax.axis_index("s")
        r0 = s * ROWS_PER_TILE
        row0 = s * nb

        def run(hp, hp2, out, col0):
            rows = pl.ds(r0, ROWS_PER_TILE)
            cols = pl.ds(col0, dh)
            pltpu.sync_copy(hp.at[rows, cols], acc_sh.at[rows])
            pltpu.sync_copy(hp.at[rows, cols], tbl_sh.at[rows])
            plsc.subcore_barrier()

            def one_path(tbl_view, src_arr):
                def wait_gather(k):
                    pltpu.make_async_copy(
                        tbl_view.at[src_v.at[0]], bufs[k], gsems[k]).wait()

                def wait_scatter(k):
                    pltpu.make_async_copy(
                        bufs[k], acc_sh.at[dst_v.at[0]], ssems[k]).wait()

                def chunk_body(ci):
                    pltpu.sync_copy(src_arr.at[pl.ds(row0 + ci * ch, ch)], src_v)
                    pltpu.sync_copy(dst2d.at[pl.ds(row0 + ci * ch, ch)], dst_v)
                    pltpu.async_copy(tbl_view.at[src_v.at[0]], bufs[0], gsems[0])
                    pltpu.async_copy(tbl_view.at[src_v.at[1]], bufs[1], gsems[1])

                    def grp_body(g):
                        for k in range(4):
                            i = g * 4 + k
                            wait_gather(k)
                            pltpu.async_copy(
                                bufs[k], acc_sh.at[dst_v.at[i]], ssems[k],
                                add=True)
                            k2 = (k + 2) % 4

                            @pl.when(i >= 2)
                            def _():
                                wait_scatter(k2)

                            @pl.when(i + 2 < ch)
                            def _():
                                pltpu.async_copy(
                                    tbl_view.at[src_v.at[i + 2]], bufs[k2],
                                    gsems[k2])

                    pl.loop(0, ngrp)(grp_body)
                    wait_scatter((ch - 2) % 4)
                    wait_scatter((ch - 1) % 4)

                pl.loop(0, nch)(chunk_body)

            @pl.when(s < NHBM)
            def _():
                one_path(hp2, srca if col0 == 0 else srcb)

            @pl.when(s >= NHBM)
            def _():
                one_path(tbl_sh, src2d)

            plsc.subcore_barrier()
            pltpu.sync_copy(acc_sh.at[rows], out.at[rows, cols])

        if narr == 1:
            @pl.when(c == 0)
            def _():
                run(hps[0], hp2s[0], outs[0], 0)

            @pl.when(c == 1)
            def _():
                run(hps[0], hp2s[0], outs[0], dh)
        else:
            half = narr // 2

            @pl.when(c == 0)
            def _():
                for p in range(half):
                    run(hps[p], hp2s[p], outs[p], 0)
                    run(hps[p], hp2s[p], outs[p], dh)

            @pl.when(c == 1)
            def _():
                for p in range(half):
                    run(hps[half + p], hp2s[half + p], outs[half + p], 0)
                    run(hps[half + p], hp2s[half + p], outs[half + p], dh)

    return agg_kernel


_R = 512
_NBLK = NP // _R


def _prep_body(deg0_ref, deg1_ref, x_ref, dis_ref, hp_ref):
    d = deg0_ref[:, 0:1] + deg1_ref[:, 0:1] - 1.0
    dis = lax.rsqrt(d)
    dis_ref[...] = dis
    hp_ref[...] = x_ref[...] * dis


def _tc_prep(deg0, deg1, x_pad):
    return pl.pallas_call(
        _prep_body,
        grid=(_NBLK,),
        in_specs=[
            pl.BlockSpec((_R, 16), lambda i: (i, 0)),
            pl.BlockSpec((_R, 16), lambda i: (i, 0)),
            pl.BlockSpec((_R, D_IN), lambda i: (i, 0)),
        ],
        out_specs=[
            pl.BlockSpec((_R, 1), lambda i: (i, 0)),
            pl.BlockSpec((_R, D_IN), lambda i: (i, 0)),
        ],
        out_shape=[
            jax.ShapeDtypeStruct((NP, 1), f32),
            jax.ShapeDtypeStruct((NP, D_IN), f32),
        ],
    )(deg0, deg1, x_pad)


def _layer_body(*refs, nin, nout, last):
    acc_refs = refs[:nin]
    dis_ref, w_ref, b_ref = refs[nin : nin + 3]
    wo_ref = refs[nin + 3] if last else None
    out_refs = refs[nin + 3 + (1 if last else 0) :]
    dis = dis_ref[...]
    z = jnp.concatenate([r[...] for r in acc_refs], axis=1) * dis
    h = jnp.dot(z, w_ref[...], preferred_element_type=f32) + b_ref[...]
    h = jnp.maximum(h, 0.0) * dis
    if last:
        out_refs[0][...] = jnp.dot(h, wo_ref[...], preferred_element_type=f32)
    else:
        w = h.shape[1] // nout
        for j, r in enumerate(out_refs):
            r[...] = h[:, j * w : (j + 1) * w]


def _tc_layer(accs, dis, w, b2d, wo_pad=None):
    din, dout = w.shape
    nin = len(accs)
    dhin = din // nin
    last = wo_pad is not None
    nout = 1 if last else dout // 128
    wout = NCP if last else 128
    in_specs = [pl.BlockSpec((_R, dhin), lambda i: (i, 0))] * nin + [
        pl.BlockSpec((_R, 1), lambda i: (i, 0)),
        pl.BlockSpec((din, dout), lambda i: (0, 0)),
        pl.BlockSpec((1, dout), lambda i: (0, 0)),
    ]
    args = list(accs) + [dis, w, b2d]
    if last:
        in_specs.append(pl.BlockSpec((NHID, NCP), lambda i: (0, 0)))
        args.append(wo_pad)
    return pl.pallas_call(
        functools.partial(_layer_body, nin=nin, nout=nout, last=last),
        grid=(_NBLK,),
        in_specs=in_specs,
        out_specs=[pl.BlockSpec((_R, wout), lambda i: (i, 0))] * nout,
        out_shape=[jax.ShapeDtypeStruct((NP, wout), f32)] * nout,
    )(*args)


def _final_body(acc_ref, dis_ref, bo_ref, out_ref):
    t = acc_ref[...] * dis_ref[...] + bo_ref[...]
    col = lax.broadcasted_iota(i32, (_R, NCP), 1)
    valid = col < NCLASS
    tm = jnp.where(valid, t, -jnp.inf)
    m = jnp.max(tm, axis=1, keepdims=True)
    e = jnp.where(valid, jnp.exp(t - m), 0.0)
    lse = jnp.log(jnp.sum(e, axis=1, keepdims=True))
    out_ref[...] = t - m - lse


def _tc_final(acc, dis, bo2d):
    return pl.pallas_call(
        _final_body,
        grid=(_NBLK,),
        in_specs=[
            pl.BlockSpec((_R, NCP), lambda i: (i, 0)),
            pl.BlockSpec((_R, 1), lambda i: (i, 0)),
            pl.BlockSpec((1, NCP), lambda i: (0, 0)),
        ],
        out_specs=pl.BlockSpec((_R, NCP), lambda i: (i, 0)),
        out_shape=jax.ShapeDtypeStruct((NP, NCP), f32),
    )(acc, dis, bo2d)


_sc_cache = {}


def _get_deg():
    if "deg" not in _sc_cache:
        _sc_cache["deg"] = _make_deg_kernel()
    return _sc_cache["deg"]


def _get_agg(dh, narr):
    if (dh, narr) not in _sc_cache:
        _sc_cache[(dh, narr)] = _make_agg_kernel(dh, narr)
    return _sc_cache[(dh, narr)]


@jax.jit
def kernel(x, edge_index, W1, b1, W2, b2, W3, b3, Wo, bo):
    ei_pad = jnp.full((2, EP), NP - 1, i32).at[:, :E].set(edge_index)
    src = ei_pad[0].reshape(EP // EB, EB)
    dst = ei_pad[1].reshape(EP // EB, EB)
    srca = (ei_pad[0] * 2).reshape(EP // EB, EB)
    srcb = (ei_pad[0] * 2 + 1).reshape(EP // EB, EB)
    x_pad = jnp.zeros((NP, D_IN), f32).at[:N].set(x)
    ones = jnp.ones((ROWS_PER_TILE, 16), f32)
    wo_pad = jnp.zeros((NHID, NCP), f32).at[:, :NCLASS].set(Wo)
    bo_pad = jnp.zeros((1, NCP), f32).at[0, :NCLASS].set(bo)

    def v2(a):
        return lax.optimization_barrier(jnp.reshape(a, (2 * NP, a.shape[1] // 2)))

    deg0, deg1 = _get_deg()(dst, ones)
    dis, hp0 = _tc_prep(deg0, deg1, x_pad)

    a0 = _get_agg(64, 1)(hp0, v2(hp0), src, srca, srcb, dst)
    h1 = _tc_layer(a0, dis, W1, b1.reshape(1, -1))

    a1 = _get_agg(64, 2)(*h1, *map(v2, h1), src, srca, srcb, dst)
    h2 = _tc_layer(a1, dis, W2, b2.reshape(1, -1))

    a2 = _get_agg(64, 2)(*h2, *map(v2, h2), src, srca, srcb, dst)
    zp = _tc_layer(a2, dis, W3, b3.reshape(1, -1), wo_pad)

    a3 = _get_agg(32, 1)(zp[0], v2(zp[0]), src, srca, srcb, dst)
    out = _tc_final(a3[0], dis, bo_pad)
    return out[:N, :NCLASS]


# speedup vs baseline: 1.2479x; 1.2479x over previous
import functools

import jax
import jax.numpy as jnp
from jax import lax
from jax.experimental import pallas as pl
from jax.experimental.pallas import tpu as pltpu
from jax.experimental.pallas import tpu_sc as plsc

N = 10000
NP = 10240
E = 320000
EP = 327680
D_IN = 128
NHID = 256
NCLASS = 40
NCP = 64

NC = 2
NS = 16
EB = 128
ROWS_PER_TILE = NP // NS

f32 = jnp.float32
i32 = jnp.int32


def _sc_mesh():
    return plsc.VectorSubcoreMesh(
        core_axis_name="c", subcore_axis_name="s", num_cores=NC, num_subcores=NS
    )


def _make_deg_kernel():
    ept = EP // (NC * NS)
    nb = ept // EB

    @functools.partial(
        pl.kernel,
        out_type=[
            jax.ShapeDtypeStruct((NP, 16), f32),
            jax.ShapeDtypeStruct((NP, 16), f32),
        ],
        mesh=_sc_mesh(),
        scratch_types=[
            pltpu.VMEM((EP // (NC * NS * EB), EB), i32),
            pltpu.VMEM((EB, 16), f32),
            pltpu.VMEM_SHARED((NP, 16), f32),
        ],
        compiler_params=pltpu.CompilerParams(use_tc_tiling_on_sc=False),
    )
    def deg_kernel(dst2d_hbm, ones_hbm, deg0_hbm, deg1_hbm, dst_v, ones_v, acc_sh):
        c = lax.axis_index("c")
        s = lax.axis_index("s")
        r0 = s * ROWS_PER_TILE
        pltpu.sync_copy(ones_hbm, acc_sh.at[pl.ds(r0, ROWS_PER_TILE)])
        pltpu.sync_copy(ones_hbm.at[pl.ds(0, EB)], ones_v)
        row0 = (c * NS + s) * nb
        pltpu.sync_copy(dst2d_hbm.at[pl.ds(row0, nb)], dst_v)
        plsc.subcore_barrier()

        def body(i):
            pltpu.sync_copy(ones_v, acc_sh.at[dst_v.at[i]], add=True)

        pl.loop(0, nb)(body)
        plsc.subcore_barrier()

        @pl.when(c == 0)
        def _():
            pltpu.sync_copy(
                acc_sh.at[pl.ds(r0, ROWS_PER_TILE)],
                deg0_hbm.at[pl.ds(r0, ROWS_PER_TILE)],
            )

        @pl.when(c == 1)
        def _():
            pltpu.sync_copy(
                acc_sh.at[pl.ds(r0, ROWS_PER_TILE)],
                deg1_hbm.at[pl.ds(r0, ROWS_PER_TILE)],
            )

    return deg_kernel


def _make_agg_kernel(dh, narr):
    ept = EP // NS
    nb = ept // EB
    ch = 32
    nch = nb // ch
    ngrp = ch // 4

    @functools.partial(
        pl.kernel,
        out_type=[jax.ShapeDtypeStruct((NP, 2 * dh), f32)] * narr,
        mesh=_sc_mesh(),
        scratch_types=[
            pltpu.VMEM((ch, EB), i32),
            pltpu.VMEM((ch, EB), i32),
            [pltpu.VMEM((EB, dh), f32)] * 4,
            pltpu.VMEM_SHARED((NP, dh), f32),
            pltpu.VMEM_SHARED((NP, dh), f32),
            [pltpu.SemaphoreType.DMA] * 4,
            [pltpu.SemaphoreType.DMA] * 4,
        ],
        compiler_params=pltpu.CompilerParams(use_tc_tiling_on_sc=False),
    )
    def agg_kernel(*refs):
        hps = refs[:narr]
        src2d, dst2d = refs[narr], refs[narr + 1]
        outs = refs[narr + 2 : 2 * narr + 2]
        src_v, dst_v, bufs, acc_sh, tbl_sh, gsems, ssems = refs[2 * narr + 2 :]
        c = lax.axis_index("c")
        s = lax.axis_index("s")
        r0 = s * ROWS_PER_TILE
        row0 = s * nb

        def run(hp, out, col0):
            rows = pl.ds(r0, ROWS_PER_TILE)
            cols = pl.ds(col0, dh)
            pltpu.sync_copy(hp.at[rows, cols], acc_sh.at[rows])
            pltpu.sync_copy(hp.at[rows, cols], tbl_sh.at[rows])
            plsc.subcore_barrier()

            def one_path(tbl_view, src_arr):
                def wait_gather(k):
                    pltpu.make_async_copy(
                        tbl_view.at[src_v.at[0]], bufs[k], gsems[k]).wait()

                def wait_scatter(k):
                    pltpu.make_async_copy(
                        bufs[k], acc_sh.at[dst_v.at[0]], ssems[k]).wait()

                def chunk_body(ci):
                    pltpu.sync_copy(src_arr.at[pl.ds(row0 + ci * ch, ch)], src_v)
                    pltpu.sync_copy(dst2d.at[pl.ds(row0 + ci * ch, ch)], dst_v)
                    pltpu.async_copy(tbl_view.at[src_v.at[0]], bufs[0], gsems[0])
                    pltpu.async_copy(tbl_view.at[src_v.at[1]], bufs[1], gsems[1])

                    def grp_body(g):
                        for k in range(4):
                            i = g * 4 + k
                            wait_gather(k)
                            pltpu.async_copy(
                                bufs[k], acc_sh.at[dst_v.at[i]], ssems[k],
                                add=True)
                            k2 = (k + 2) % 4

                            @pl.when(i >= 2)
                            def _():
                                wait_scatter(k2)

                            @pl.when(i + 2 < ch)
                            def _():
                                pltpu.async_copy(
                                    tbl_view.at[src_v.at[i + 2]], bufs[k2],
                                    gsems[k2])

                    pl.loop(0, ngrp)(grp_body)
                    wait_scatter((ch - 2) % 4)
                    wait_scatter((ch - 1) % 4)

                pl.loop(0, nch)(chunk_body)

            one_path(tbl_sh, src2d)
            plsc.subcore_barrier()
            pltpu.sync_copy(acc_sh.at[rows], out.at[rows, cols])

        if narr == 1:
            @pl.when(c == 0)
            def _():
                run(hps[0], outs[0], 0)

            @pl.when(c == 1)
            def _():
                run(hps[0], outs[0], dh)
        else:
            half = narr // 2

            @pl.when(c == 0)
            def _():
                for p in range(half):
                    run(hps[p], outs[p], 0)
                    run(hps[p], outs[p], dh)

            @pl.when(c == 1)
            def _():
                for p in range(half):
                    run(hps[half + p], outs[half + p], 0)
                    run(hps[half + p], outs[half + p], dh)

    return agg_kernel


_R = 512
_NBLK = NP // _R


def _prep_body(deg0_ref, deg1_ref, x_ref, dis_ref, hp_ref):
    d = deg0_ref[:, 0:1] + deg1_ref[:, 0:1] - 1.0
    dis = lax.rsqrt(d)
    dis_ref[...] = dis
    hp_ref[...] = x_ref[...] * dis


def _tc_prep(deg0, deg1, x_pad):
    return pl.pallas_call(
        _prep_body,
        grid=(_NBLK,),
        in_specs=[
            pl.BlockSpec((_R, 16), lambda i: (i, 0)),
            pl.BlockSpec((_R, 16), lambda i: (i, 0)),
            pl.BlockSpec((_R, D_IN), lambda i: (i, 0)),
        ],
        out_specs=[
            pl.BlockSpec((_R, 1), lambda i: (i, 0)),
            pl.BlockSpec((_R, D_IN), lambda i: (i, 0)),
        ],
        out_shape=[
            jax.ShapeDtypeStruct((NP, 1), f32),
            jax.ShapeDtypeStruct((NP, D_IN), f32),
        ],
    )(deg0, deg1, x_pad)


def _layer_body(*refs, nin, nout, last):
    acc_refs = refs[:nin]
    dis_ref, w_ref, b_ref = refs[nin : nin + 3]
    wo_ref = refs[nin + 3] if last else None
    out_refs = refs[nin + 3 + (1 if last else 0) :]
    dis = dis_ref[...]
    z = jnp.concatenate([r[...] for r in acc_refs], axis=1) * dis
    h = jnp.dot(z, w_ref[...], preferred_element_type=f32) + b_ref[...]
    h = jnp.maximum(h, 0.0) * dis
    if last:
        out_refs[0][...] = jnp.dot(h, wo_ref[...], preferred_element_type=f32)
    else:
        w = h.shape[1] // nout
        for j, r in enumerate(out_refs):
            r[...] = h[:, j * w : (j + 1) * w]


def _tc_layer(accs, dis, w, b2d, wo_pad=None):
    din, dout = w.shape
    nin = len(accs)
    dhin = din // nin
    last = wo_pad is not None
    nout = 1 if last else dout // 128
    wout = NCP if last else 128
    in_specs = [pl.BlockSpec((_R, dhin), lambda i: (i, 0))] * nin + [
        pl.BlockSpec((_R, 1), lambda i: (i, 0)),
        pl.BlockSpec((din, dout), lambda i: (0, 0)),
        pl.BlockSpec((1, dout), lambda i: (0, 0)),
    ]
    args = list(accs) + [dis, w, b2d]
    if last:
        in_specs.append(pl.BlockSpec((NHID, NCP), lambda i: (0, 0)))
        args.append(wo_pad)
    return pl.pallas_call(
        functools.partial(_layer_body, nin=nin, nout=nout, last=last),
        grid=(_NBLK,),
        in_specs=in_specs,
        out_specs=[pl.BlockSpec((_R, wout), lambda i: (i, 0))] * nout,
        out_shape=[jax.ShapeDtypeStruct((NP, wout), f32)] * nout,
    )(*args)


def _final_body(acc_ref, dis_ref, bo_ref, out_ref):
    t = acc_ref[...] * dis_ref[...] + bo_ref[...]
    col = lax.broadcasted_iota(i32, (_R, NCP), 1)
    valid = col < NCLASS
    tm = jnp.where(valid, t, -jnp.inf)
    m = jnp.max(tm, axis=1, keepdims=True)
    e = jnp.where(valid, jnp.exp(t - m), 0.0)
    lse = jnp.log(jnp.sum(e, axis=1, keepdims=True))
    out_ref[...] = t - m - lse


def _tc_final(acc, dis, bo2d):
    return pl.pallas_call(
        _final_body,
        grid=(_NBLK,),
        in_specs=[
            pl.BlockSpec((_R, NCP), lambda i: (i, 0)),
            pl.BlockSpec((_R, 1), lambda i: (i, 0)),
            pl.BlockSpec((1, NCP), lambda i: (0, 0)),
        ],
        out_specs=pl.BlockSpec((_R, NCP), lambda i: (i, 0)),
        out_shape=jax.ShapeDtypeStruct((NP, NCP), f32),
    )(acc, dis, bo2d)


_sc_cache = {}


def _get_deg():
    if "deg" not in _sc_cache:
        _sc_cache["deg"] = _make_deg_kernel()
    return _sc_cache["deg"]


def _get_agg(dh, narr):
    if (dh, narr) not in _sc_cache:
        _sc_cache[(dh, narr)] = _make_agg_kernel(dh, narr)
    return _sc_cache[(dh, narr)]


@jax.jit
def kernel(x, edge_index, W1, b1, W2, b2, W3, b3, Wo, bo):
    ei_pad = jnp.full((2, EP), NP - 1, i32).at[:, :E].set(edge_index)
    src = ei_pad[0].reshape(EP // EB, EB)
    dst = ei_pad[1].reshape(EP // EB, EB)
    x_pad = jnp.zeros((NP, D_IN), f32).at[:N].set(x)
    ones = jnp.ones((ROWS_PER_TILE, 16), f32)
    wo_pad = jnp.zeros((NHID, NCP), f32).at[:, :NCLASS].set(Wo)
    bo_pad = jnp.zeros((1, NCP), f32).at[0, :NCLASS].set(bo)

    deg0, deg1 = _get_deg()(dst, ones)
    dis, hp0 = _tc_prep(deg0, deg1, x_pad)

    a0 = _get_agg(64, 1)(hp0, src, dst)
    h1 = _tc_layer(a0, dis, W1, b1.reshape(1, -1))

    a1 = _get_agg(64, 2)(*h1, src, dst)
    h2 = _tc_layer(a1, dis, W2, b2.reshape(1, -1))

    a2 = _get_agg(64, 2)(*h2, src, dst)
    zp = _tc_layer(a2, dis, W3, b3.reshape(1, -1), wo_pad)

    a3 = _get_agg(32, 1)(zp[0], src, dst)
    out = _tc_final(a3[0], dis, bo_pad)
    return out[:N, :NCLASS]
